# baseline (speedup 1.0000x reference)
import functools

import jax
import jax.numpy as jnp
from jax import lax
from jax.experimental import pallas as pl
from jax.experimental.pallas import tpu as pltpu
from jax.experimental.pallas import tpu_sc as plsc

_NC = 2
_NS = 16
_NW = _NC * _NS

_N_SC = 8192
_TC_BLOCK = 1024
_R = 32


def _tc_add(x_ref, e_ref, o_ref):
    o_ref[...] = x_ref[...] + e_ref[...]


def _make_sc_add(total_rows, t, d, n_sc):
    rw = n_sc // _NW
    g_tiles = rw // _R
    n_tc = total_rows - n_sc
    mesh = plsc.VectorSubcoreMesh(
        core_axis_name="c", subcore_axis_name="s",
        num_cores=_NC, num_subcores=_NS,
    )

    @functools.partial(
        pl.kernel,
        out_type=jax.ShapeDtypeStruct((n_sc * d,), jnp.float32),
        mesh=mesh,
        scratch_types=[
            pltpu.VMEM((_R * d,), jnp.float32),
            pltpu.VMEM((_R * d,), jnp.float32),
        ],
    )
    def sc_add(x_hbm, e_hbm, o_hbm, xb, eb):
        wid = lax.axis_index("s") * _NC + lax.axis_index("c")
        base = n_tc + wid * rw

        def tile(g, _):
            a = base + g * _R
            ebase = lax.rem(a, t) * d
            pltpu.sync_copy(x_hbm.at[pl.ds(a * d, _R * d)], xb)
            pltpu.sync_copy(e_hbm.at[pl.ds(ebase, _R * d)], eb)

            @plsc.parallel_loop(0, _R * d, 16, unroll=8)
            def _(k):
                xb[pl.ds(k, 16)] = xb[pl.ds(k, 16)] + eb[pl.ds(k, 16)]

            pltpu.sync_copy(xb, o_hbm.at[pl.ds((a - n_tc) * d, _R * d)])
            return 0

        lax.fori_loop(0, g_tiles, tile, 0)

    return sc_add


def kernel(x, embeddings):
    b, t, d = x.shape
    total = b * t
    n_sc = _N_SC
    n_tc = total - n_sc
    emb = embeddings[:t]

    xf = x.reshape(total, d)
    x1 = x.reshape(total * d)
    e1 = emb.reshape(t * d)

    sc_out = _make_sc_add(total, t, d, n_sc)(x1, e1)

    blocks_per_seq = t // _TC_BLOCK
    n_batch_tc = n_tc // t
    tc_out = pl.pallas_call(
        _tc_add,
        grid=(blocks_per_seq, n_batch_tc),
        in_specs=[
            pl.BlockSpec((_TC_BLOCK, d), lambda j, i: (i * blocks_per_seq + j, 0)),
            pl.BlockSpec((_TC_BLOCK, d), lambda j, i: (j, 0)),
        ],
        out_specs=pl.BlockSpec((_TC_BLOCK, d), lambda j, i: (i * blocks_per_seq + j, 0)),
        out_shape=jax.ShapeDtypeStruct((n_tc, d), x.dtype),
    )(xf, emb)

    out = jnp.concatenate([tc_out, sc_out.reshape(n_sc, d)], axis=0)
    return out.reshape(b, t, d)


# speedup vs baseline: 3.7977x; 3.7977x over previous
import jax
import jax.numpy as jnp
from jax.experimental import pallas as pl

_S = 1024


def _add_kernel(x_ref, e_ref, o_ref):
    o_ref[...] = x_ref[...] + e_ref[...]


def kernel(x, embeddings):
    b, t, d = x.shape
    emb = embeddings[:t]
    xf = x.reshape(b * t, d)
    nseq = t // _S
    out = pl.pallas_call(
        _add_kernel,
        grid=(nseq, b),
        in_specs=[
            pl.BlockSpec((_S, d), lambda j, i: (i * nseq + j, 0)),
            pl.BlockSpec((_S, d), lambda j, i: (j, 0)),
        ],
        out_specs=pl.BlockSpec((_S, d), lambda j, i: (i * nseq + j, 0)),
        out_shape=jax.ShapeDtypeStruct((b * t, d), x.dtype),
    )(xf, emb)
    return out.reshape(b, t, d)


# speedup vs baseline: 3.9267x; 1.0340x over previous
import jax
import jax.numpy as jnp
from jax.experimental import pallas as pl

_SEQ_BLOCK = 512


def _add_kernel(x_ref, e_ref, o_ref):
    o_ref[...] = x_ref[...] + e_ref[...][None, :, :]


def kernel(x, embeddings):
    b, t, d = x.shape
    emb = embeddings[:t]
    return pl.pallas_call(
        _add_kernel,
        grid=(t // _SEQ_BLOCK,),
        in_specs=[
            pl.BlockSpec((b, _SEQ_BLOCK, d), lambda i: (0, i, 0)),
            pl.BlockSpec((_SEQ_BLOCK, d), lambda i: (i, 0)),
        ],
        out_specs=pl.BlockSpec((b, _SEQ_BLOCK, d), lambda i: (0, i, 0)),
        out_shape=jax.ShapeDtypeStruct(x.shape, x.dtype),
    )(x, emb)


# speedup vs baseline: 3.9457x; 1.0048x over previous
import jax
import jax.numpy as jnp
from jax.experimental import pallas as pl

_SEQ_BLOCK = 512


def _add_kernel(x_ref, e_ref, o_ref):
    o_ref[...] = x_ref[...] + e_ref[...][None, :, :]


def kernel(x, embeddings):
    b, t, d = x.shape
    emb = embeddings[:t]
    return pl.pallas_call(
        _add_kernel,
        grid=(t // 1024, b // 2),
        in_specs=[
            pl.BlockSpec((2, 1024, d), lambda j, i: (i, j, 0)),
            pl.BlockSpec((1024, d), lambda j, i: (j, 0)),
        ],
        out_specs=pl.BlockSpec((2, 1024, d), lambda j, i: (i, j, 0)),
        out_shape=jax.ShapeDtypeStruct(x.shape, x.dtype),
    )(x, emb)


# speedup vs baseline: 3.9579x; 1.0031x over previous
import jax
import jax.numpy as jnp
from jax.experimental import pallas as pl

_SEQ_BLOCK = 512


def _add_kernel(x_ref, e_ref, o_ref):
    o_ref[...] = x_ref[...] + e_ref[...][None, :, :]


def kernel(x, embeddings):
    b, t, d = x.shape
    emb = embeddings[:t]
    return pl.pallas_call(
        _add_kernel,
        grid=(t // 2048, b),
        in_specs=[
            pl.BlockSpec((1, 2048, d), lambda j, i: (i, j, 0)),
            pl.BlockSpec((2048, d), lambda j, i: (j, 0)),
        ],
        out_specs=pl.BlockSpec((1, 2048, d), lambda j, i: (i, j, 0)),
        out_shape=jax.ShapeDtypeStruct(x.shape, x.dtype),
    )(x, emb)
